# Initial kernel scaffold; baseline (speedup 1.0000x reference)
import functools

import jax
import jax.numpy as jnp
from jax import lax
from jax.experimental import pallas as pl
from jax.experimental.pallas import tpu as pltpu
from jax.experimental.pallas import tpu_sc as plsc

_N = 10000
_P = 320000
_D = 128
_NC = 2
_NS = 16
_NL = 16
_NW = _NC * _NS
_GSZ = 128
_NG = _P // _GSZ
_GPT = _NG // _NW
_REM = _NG - _GPT * _NW
_RPS = _N // _NS
_HW = 16

_mesh = plsc.VectorSubcoreMesh(core_axis_name="c", subcore_axis_name="s")


def _fill(buf, nrows, ncols, value):
    vv = jnp.full((_NL,), value, jnp.float32)

    @pl.loop(0, nrows)
    def _(i):
        @pl.loop(0, ncols, step=_NL)
        def _(k):
            buf[i, pl.ds(k, _NL)] = vv


def _zero_stripe(acc_sh, zbuf, row_base):
    for r in range(0, _RPS - _GSZ + 1, _GSZ):
        pltpu.sync_copy(zbuf, acc_sh.at[pl.ds(row_base + r, _GSZ)])
    tail = _RPS % _GSZ
    if tail:
        pltpu.sync_copy(zbuf.at[pl.ds(0, tail)],
                        acc_sh.at[pl.ds(row_base + _RPS - tail, tail)])


def _spmm_body(src_hbm, dst_hbm, table_hbm, out_hbm,
               srcidx_v, dstidx_v, rows_a, rows_b, tsrc_v, tdst_v,
               acc_sh, sem_a, sem_b):
    c = lax.axis_index("c")
    s = lax.axis_index("s")
    wid = s * _NC + c
    row_base = s * _RPS

    _fill(rows_a, _GSZ, _D, 0.0)
    _zero_stripe(acc_sh, rows_a, row_base)
    plsc.subcore_barrier()

    pltpu.sync_copy(src_hbm.at[pl.ds(wid * _GPT, _GPT)], srcidx_v)
    pltpu.sync_copy(dst_hbm.at[pl.ds(wid * _GPT, _GPT)], dstidx_v)

    bufs = (rows_a, rows_b)
    sems = (sem_a, sem_b)
    copies = [None, None]
    copies[0] = pltpu.async_copy(table_hbm.at[srcidx_v.at[0]], bufs[0], sems[0])
    for j in range(_GPT):
        cur = j % 2
        if j + 1 < _GPT:
            nxt = (j + 1) % 2
            copies[nxt] = pltpu.async_copy(
                table_hbm.at[srcidx_v.at[j + 1]], bufs[nxt], sems[nxt])
        copies[cur].wait()
        pltpu.sync_copy(bufs[cur], acc_sh.at[dstidx_v.at[j]], add=True)

    @pl.when(wid < _REM)
    def _():
        g = _GPT * _NW + wid
        pltpu.sync_copy(src_hbm.at[pl.ds(g, 1)], tsrc_v)
        pltpu.sync_copy(dst_hbm.at[pl.ds(g, 1)], tdst_v)
        pltpu.sync_copy(table_hbm.at[tsrc_v.at[0]], rows_a)
        pltpu.sync_copy(rows_a, acc_sh.at[tdst_v.at[0]], add=True)

    plsc.subcore_barrier()
    pltpu.sync_copy(acc_sh.at[pl.ds(row_base, _RPS)],
                    out_hbm.at[c, pl.ds(row_base, _RPS)])


@functools.partial(
    pl.kernel, mesh=_mesh,
    out_type=jax.ShapeDtypeStruct((_NC, _N, _D), jnp.float32),
    scratch_types=[
        pltpu.VMEM((_GPT, _GSZ), jnp.int32),
        pltpu.VMEM((_GPT, _GSZ), jnp.int32),
        pltpu.VMEM((_GSZ, _D), jnp.float32),
        pltpu.VMEM((_GSZ, _D), jnp.float32),
        pltpu.VMEM((1, _GSZ), jnp.int32),
        pltpu.VMEM((1, _GSZ), jnp.int32),
        pltpu.VMEM_SHARED((_N, _D), jnp.float32),
        pltpu.SemaphoreType.DMA,
        pltpu.SemaphoreType.DMA,
    ],
)
def _sc_spmm(*args):
    _spmm_body(*args)


def _deg_body(nidx_hbm, eidx_hbm, outn_hbm, oute_hbm,
              nidx_v, eidx_v, ones_v, zbuf, tn_v, te_v,
              histn_sh, histe_sh):
    c = lax.axis_index("c")
    s = lax.axis_index("s")
    wid = s * _NC + c
    row_base = s * _RPS

    _fill(zbuf, _GSZ, _HW, 0.0)
    _fill(ones_v, _GSZ, _HW, 1.0)
    _zero_stripe(histn_sh, zbuf, row_base)
    _zero_stripe(histe_sh, zbuf, row_base)
    plsc.subcore_barrier()

    pltpu.sync_copy(nidx_hbm.at[pl.ds(wid * _GPT, _GPT)], nidx_v)
    pltpu.sync_copy(eidx_hbm.at[pl.ds(wid * _GPT, _GPT)], eidx_v)

    for j in range(_GPT):
        pltpu.sync_copy(ones_v, histn_sh.at[nidx_v.at[j]], add=True)
        pltpu.sync_copy(ones_v, histe_sh.at[eidx_v.at[j]], add=True)

    @pl.when(wid < _REM)
    def _():
        g = _GPT * _NW + wid
        pltpu.sync_copy(nidx_hbm.at[pl.ds(g, 1)], tn_v)
        pltpu.sync_copy(eidx_hbm.at[pl.ds(g, 1)], te_v)
        pltpu.sync_copy(ones_v, histn_sh.at[tn_v.at[0]], add=True)
        pltpu.sync_copy(ones_v, histe_sh.at[te_v.at[0]], add=True)

    plsc.subcore_barrier()
    pltpu.sync_copy(histn_sh.at[pl.ds(row_base, _RPS)],
                    outn_hbm.at[c, pl.ds(row_base, _RPS)])
    pltpu.sync_copy(histe_sh.at[pl.ds(row_base, _RPS)],
                    oute_hbm.at[c, pl.ds(row_base, _RPS)])


@functools.partial(
    pl.kernel, mesh=_mesh,
    out_type=[jax.ShapeDtypeStruct((_NC, _N, _HW), jnp.float32),
              jax.ShapeDtypeStruct((_NC, _N, _HW), jnp.float32)],
    scratch_types=[
        pltpu.VMEM((_GPT, _GSZ), jnp.int32),
        pltpu.VMEM((_GPT, _GSZ), jnp.int32),
        pltpu.VMEM((_GSZ, _HW), jnp.float32),
        pltpu.VMEM((_GSZ, _HW), jnp.float32),
        pltpu.VMEM((1, _GSZ), jnp.int32),
        pltpu.VMEM((1, _GSZ), jnp.int32),
        pltpu.VMEM_SHARED((_N, _HW), jnp.float32),
        pltpu.VMEM_SHARED((_N, _HW), jnp.float32),
    ],
)
def _sc_degrees(*args):
    _deg_body(*args)


_BLK = 2000


def _mm_body(x_ref, w_ref, o_ref):
    o_ref[...] = jnp.dot(x_ref[...], w_ref[...],
                         preferred_element_type=jnp.float32)


def _tc_matmul(x, w):
    return pl.pallas_call(
        _mm_body,
        grid=(_N // _BLK,),
        in_specs=[pl.BlockSpec((_BLK, _D), lambda i: (i, 0)),
                  pl.BlockSpec((_D, _D), lambda i: (0, 0))],
        out_specs=pl.BlockSpec((_BLK, _D), lambda i: (i, 0)),
        out_shape=jax.ShapeDtypeStruct((_N, _D), jnp.float32),
    )(x, w)


def _recip_deg(h_ref):
    hv = h_ref[...]
    d = hv[0, :, 0:1] + hv[1, :, 0:1]
    return jnp.where(d > 0, 1.0 / d, 0.0)


def _scale_body(p_ref, h_ref, o_ref):
    o_ref[...] = (p_ref[0] + p_ref[1]) * _recip_deg(h_ref)


def _tc_scale(parts, hist):
    return pl.pallas_call(
        _scale_body,
        grid=(_N // _BLK,),
        in_specs=[pl.BlockSpec((_NC, _BLK, _D), lambda i: (0, i, 0)),
                  pl.BlockSpec((_NC, _BLK, _HW), lambda i: (0, i, 0))],
        out_specs=pl.BlockSpec((_BLK, _D), lambda i: (i, 0)),
        out_shape=jax.ShapeDtypeStruct((_N, _D), jnp.float32),
    )(parts, hist)


def _mid_body(p_ref, h_ref, b_ref, w_ref, o_ref):
    acc = (p_ref[0] + p_ref[1]) * _recip_deg(h_ref) + b_ref[...]
    hmid = jnp.maximum(acc, 0.0)
    o_ref[...] = jnp.dot(hmid, w_ref[...], preferred_element_type=jnp.float32)


def _tc_mid(parts, hist, b, w):
    return pl.pallas_call(
        _mid_body,
        grid=(_N // _BLK,),
        in_specs=[pl.BlockSpec((_NC, _BLK, _D), lambda i: (0, i, 0)),
                  pl.BlockSpec((_NC, _BLK, _HW), lambda i: (0, i, 0)),
                  pl.BlockSpec((1, _D), lambda i: (0, 0)),
                  pl.BlockSpec((_D, _D), lambda i: (0, 0))],
        out_specs=pl.BlockSpec((_BLK, _D), lambda i: (i, 0)),
        out_shape=jax.ShapeDtypeStruct((_N, _D), jnp.float32),
    )(parts, hist, b, w)


def _fin_body(p_ref, h_ref, b_ref, o_ref):
    o_ref[...] = (p_ref[0] + p_ref[1]) * _recip_deg(h_ref) + b_ref[...]


def _tc_final(parts, hist, b):
    return pl.pallas_call(
        _fin_body,
        grid=(_N // _BLK,),
        in_specs=[pl.BlockSpec((_NC, _BLK, _D), lambda i: (0, i, 0)),
                  pl.BlockSpec((_NC, _BLK, _HW), lambda i: (0, i, 0)),
                  pl.BlockSpec((1, _D), lambda i: (0, 0))],
        out_specs=pl.BlockSpec((_BLK, _D), lambda i: (i, 0)),
        out_shape=jax.ShapeDtypeStruct((_N, _D), jnp.float32),
    )(parts, hist, b)


def kernel(x, edge_index, W1, b1, W2, b2):
    node_g = edge_index[0].reshape(_NG, _GSZ)
    hedge_g = edge_index[1].reshape(_NG, _GSZ)
    b1r = b1.reshape(1, _D)
    b2r = b2.reshape(1, _D)

    hist_n, hist_e = _sc_degrees(node_g, hedge_g)

    xw1 = _tc_matmul(x, W1)
    ep1 = _sc_spmm(node_g, hedge_g, xw1)
    ef1 = _tc_scale(ep1, hist_e)
    np1 = _sc_spmm(hedge_g, node_g, ef1)
    xw2 = _tc_mid(np1, hist_n, b1r, W2)
    ep2 = _sc_spmm(node_g, hedge_g, xw2)
    ef2 = _tc_scale(ep2, hist_e)
    np2 = _sc_spmm(hedge_g, node_g, ef2)
    out = _tc_final(np2, hist_n, b2r)
    return out


# speedup vs baseline: 12.0235x; 12.0235x over previous
import functools

import jax
import jax.numpy as jnp
from jax import lax
from jax.experimental import pallas as pl
from jax.experimental.pallas import tpu as pltpu
from jax.experimental.pallas import tpu_sc as plsc

_N = 10000
_P = 320000
_D = 128
_NC = 2
_NS = 16
_NL = 16
_NW = _NC * _NS
_GSZ = 128
_NG = 2560
_PP = _NG * _GSZ
_GPT = _NG // _NW
_IHALF = _GPT // 2
_NPAD = 16
_NA = _N + _NPAD
_SPR = 624
_LAST = _NA - _SPR * _NS
_HW = 16

_mesh = plsc.VectorSubcoreMesh(core_axis_name="c", subcore_axis_name="s")


def _fill(buf, nrows, ncols, value):
    vv = jnp.full((_NL,), value, jnp.float32)

    @pl.loop(0, nrows)
    def _(i):
        @pl.loop(0, ncols, step=_NL)
        def _(k):
            buf[i, pl.ds(k, _NL)] = vv


def _stripe_copy(src_at, dst_at, rows):
    off = 0
    while off + _GSZ <= rows:
        pltpu.sync_copy(src_at(off, _GSZ), dst_at(off, _GSZ))
        off += _GSZ
    if off < rows:
        pltpu.sync_copy(src_at(off, rows - off), dst_at(off, rows - off))


def _zero_stripes(acc_sh, zbuf, s):
    row_base = pl.multiple_of(s * _SPR, 8)
    _stripe_copy(lambda o, n: zbuf.at[pl.ds(0, n)],
                 lambda o, n: acc_sh.at[pl.ds(row_base + o, n)], _SPR)

    @pl.when(s == _NS - 1)
    def _():
        pltpu.sync_copy(zbuf.at[pl.ds(0, _LAST)],
                        acc_sh.at[pl.ds(_SPR * _NS, _LAST)])


def _writeback_stripes(acc_sh, out_hbm, c, s):
    row_base = pl.multiple_of(s * _SPR, 8)
    _stripe_copy(lambda o, n: acc_sh.at[pl.ds(row_base + o, n)],
                 lambda o, n: out_hbm.at[c, pl.ds(row_base + o, n)], _SPR)

    @pl.when(s == _NS - 1)
    def _():
        pltpu.sync_copy(acc_sh.at[pl.ds(_SPR * _NS, _N - _SPR * _NS)],
                        out_hbm.at[c, pl.ds(_SPR * _NS, _N - _SPR * _NS)])


def _spmm_body(src_hbm, dst_hbm, table_hbm, out_hbm,
               srcidx_v, dstidx_v, rows_a, rows_b,
               acc_sh, sem_a, sem_b):
    c = lax.axis_index("c")
    s = lax.axis_index("s")
    wid = s * _NC + c

    _fill(rows_a, _GSZ, _D, 0.0)
    _zero_stripes(acc_sh, rows_a, s)
    plsc.subcore_barrier()

    def body(si_vmem, di_vmem):
        pltpu.sync_copy(table_hbm.at[si_vmem.at[0]], rows_a)
        pltpu.sync_copy(rows_a, acc_sh.at[di_vmem.at[0]], add=True)

    gidx = lambda k: (0, wid * _GPT + k)
    pltpu.emit_pipeline(
        body,
        grid=(_GPT,),
        in_specs=[pl.BlockSpec((1, _GSZ), gidx),
                  pl.BlockSpec((1, _GSZ), gidx)],
        out_specs=[],
        dimension_semantics=(pltpu.ARBITRARY,),
    )(src_hbm, dst_hbm)

    plsc.subcore_barrier()
    _writeback_stripes(acc_sh, out_hbm, c, s)


@functools.partial(
    pl.kernel, mesh=_mesh,
    out_type=jax.ShapeDtypeStruct((_NC, _N, _D), jnp.float32),
    scratch_types=[
        pltpu.VMEM((_IHALF * _GSZ,), jnp.int32),
        pltpu.VMEM((_IHALF * _GSZ,), jnp.int32),
        pltpu.VMEM((_GSZ, _D), jnp.float32),
        pltpu.VMEM((_GSZ, _D), jnp.float32),
        pltpu.VMEM_SHARED((_NA, _D), jnp.float32),
        pltpu.SemaphoreType.DMA,
        pltpu.SemaphoreType.DMA,
    ],
)
def _sc_spmm(*args):
    _spmm_body(*args)


def _hist_body(idx_hbm, out_hbm, buf, hist_sh):
    c = lax.axis_index("c")
    s = lax.axis_index("s")
    wid = s * _NC + c

    _fill(buf, _GSZ, _D, 0.0)
    _zero_stripes(hist_sh, buf, s)
    _fill(buf, _GSZ, _D, 1.0)
    plsc.subcore_barrier()

    def body(i_vmem):
        pltpu.sync_copy(buf, hist_sh.at[i_vmem.at[0]], add=True)

    gidx = lambda k: (0, wid * _GPT + k)
    pltpu.emit_pipeline(
        body,
        grid=(_GPT,),
        in_specs=[pl.BlockSpec((1, _GSZ), gidx)],
        out_specs=[],
        dimension_semantics=(pltpu.ARBITRARY,),
    )(idx_hbm)

    plsc.subcore_barrier()
    _writeback_stripes(hist_sh, out_hbm, c, s)


@functools.partial(
    pl.kernel, mesh=_mesh,
    out_type=jax.ShapeDtypeStruct((_NC, _N, _D), jnp.float32),
    scratch_types=[
        pltpu.VMEM((_GSZ, _D), jnp.float32),
        pltpu.VMEM_SHARED((_NA, _D), jnp.float32),
    ],
)
def _sc_hist(*args):
    _hist_body(*args)


_BLK = 2000


def _mm_body(x_ref, w_ref, o_ref):
    o_ref[...] = jnp.dot(x_ref[...], w_ref[...],
                         preferred_element_type=jnp.float32)


def _tc_matmul(x, w):
    return pl.pallas_call(
        _mm_body,
        grid=(_N // _BLK,),
        in_specs=[pl.BlockSpec((_BLK, _D), lambda i: (i, 0)),
                  pl.BlockSpec((_D, _D), lambda i: (0, 0))],
        out_specs=pl.BlockSpec((_BLK, _D), lambda i: (i, 0)),
        out_shape=jax.ShapeDtypeStruct((_N, _D), jnp.float32),
    )(x, w)


def _recip_deg(h_ref):
    hv = h_ref[...]
    d = hv[0, :, 0:1] + hv[1, :, 0:1]
    return jnp.where(d > 0, 1.0 / d, 0.0)


def _scale_body(p_ref, h_ref, o_ref):
    o_ref[...] = (p_ref[0] + p_ref[1]) * _recip_deg(h_ref)


def _tc_scale(parts, hist):
    return pl.pallas_call(
        _scale_body,
        grid=(_N // _BLK,),
        in_specs=[pl.BlockSpec((_NC, _BLK, _D), lambda i: (0, i, 0)),
                  pl.BlockSpec((_NC, _BLK, _D), lambda i: (0, i, 0))],
        out_specs=pl.BlockSpec((_BLK, _D), lambda i: (i, 0)),
        out_shape=jax.ShapeDtypeStruct((_N, _D), jnp.float32),
    )(parts, hist)


def _mid_body(p_ref, h_ref, b_ref, w_ref, o_ref):
    acc = (p_ref[0] + p_ref[1]) * _recip_deg(h_ref) + b_ref[...]
    hmid = jnp.maximum(acc, 0.0)
    o_ref[...] = jnp.dot(hmid, w_ref[...], preferred_element_type=jnp.float32)


def _tc_mid(parts, hist, b, w):
    return pl.pallas_call(
        _mid_body,
        grid=(_N // _BLK,),
        in_specs=[pl.BlockSpec((_NC, _BLK, _D), lambda i: (0, i, 0)),
                  pl.BlockSpec((_NC, _BLK, _D), lambda i: (0, i, 0)),
                  pl.BlockSpec((1, _D), lambda i: (0, 0)),
                  pl.BlockSpec((_D, _D), lambda i: (0, 0))],
        out_specs=pl.BlockSpec((_BLK, _D), lambda i: (i, 0)),
        out_shape=jax.ShapeDtypeStruct((_N, _D), jnp.float32),
    )(parts, hist, b, w)


def _fin_body(p_ref, h_ref, b_ref, o_ref):
    o_ref[...] = (p_ref[0] + p_ref[1]) * _recip_deg(h_ref) + b_ref[...]


def _tc_final(parts, hist, b):
    return pl.pallas_call(
        _fin_body,
        grid=(_N // _BLK,),
        in_specs=[pl.BlockSpec((_NC, _BLK, _D), lambda i: (0, i, 0)),
                  pl.BlockSpec((_NC, _BLK, _D), lambda i: (0, i, 0)),
                  pl.BlockSpec((1, _D), lambda i: (0, 0))],
        out_specs=pl.BlockSpec((_BLK, _D), lambda i: (i, 0)),
        out_shape=jax.ShapeDtypeStruct((_N, _D), jnp.float32),
    )(parts, hist, b)


def kernel(x, edge_index, W1, b1, W2, b2):
    npad = _PP - _P
    src_pad = (jnp.arange(npad, dtype=jnp.int32) * 61) % _N
    dst_pad = _N + (jnp.arange(npad, dtype=jnp.int32) % _NPAD)
    node_g = jnp.concatenate([edge_index[0], src_pad])
    hedge_g = jnp.concatenate([edge_index[1], src_pad])
    node_d = jnp.concatenate([edge_index[0], dst_pad])
    hedge_d = jnp.concatenate([edge_index[1], dst_pad])
    b1r = b1.reshape(1, _D)
    b2r = b2.reshape(1, _D)

    node_g = node_g.reshape(1, _PP)
    hedge_g = hedge_g.reshape(1, _PP)
    node_d = node_d.reshape(1, _PP)
    hedge_d = hedge_d.reshape(1, _PP)

    hist_n = _sc_hist(node_d)
    dep = (hist_n[0, 0, 0] * 0.0).astype(jnp.int32)
    hist_e = _sc_hist(hedge_d + dep)

    xw1 = _tc_matmul(x, W1)
    xw1 = xw1 + 0.0 * hist_e[0, 0, 0]
    ep1 = _sc_spmm(node_g, hedge_d, xw1)
    ef1 = _tc_scale(ep1, hist_e)
    np1 = _sc_spmm(hedge_g, node_d, ef1)
    xw2 = _tc_mid(np1, hist_n, b1r, W2)
    ep2 = _sc_spmm(node_g, hedge_d, xw2)
    ef2 = _tc_scale(ep2, hist_e)
    np2 = _sc_spmm(hedge_g, node_d, ef2)
    out = _tc_final(np2, hist_n, b2r)
    return out


# speedup vs baseline: 13.5532x; 1.1272x over previous
import functools

import jax
import jax.numpy as jnp
from jax import lax
from jax.experimental import pallas as pl
from jax.experimental.pallas import tpu as pltpu
from jax.experimental.pallas import tpu_sc as plsc

_N = 10000
_P = 320000
_D = 128
_NC = 2
_NS = 16
_NL = 16
_NW = _NC * _NS
_GSZ = 128
_NG = 2560
_PP = _NG * _GSZ
_GPT = _NG // _NW
_IHALF = _GPT // 2
_NPAD = 16
_NA = _N + _NPAD
_SPR = 624
_LAST = _NA - _SPR * _NS
_HW = 16

_mesh = plsc.VectorSubcoreMesh(core_axis_name="c", subcore_axis_name="s")


def _fill(buf, nrows, ncols, value):
    vv = jnp.full((_NL,), value, jnp.float32)

    @pl.loop(0, nrows)
    def _(i):
        @pl.loop(0, ncols, step=_NL)
        def _(k):
            buf[i, pl.ds(k, _NL)] = vv


def _stripe_copy(src_at, dst_at, rows):
    off = 0
    while off + _GSZ <= rows:
        pltpu.sync_copy(src_at(off, _GSZ), dst_at(off, _GSZ))
        off += _GSZ
    if off < rows:
        pltpu.sync_copy(src_at(off, rows - off), dst_at(off, rows - off))


def _zero_stripes(acc_sh, zbuf, s):
    row_base = pl.multiple_of(s * _SPR, 8)
    _stripe_copy(lambda o, n: zbuf.at[pl.ds(0, n)],
                 lambda o, n: acc_sh.at[pl.ds(row_base + o, n)], _SPR)

    @pl.when(s == _NS - 1)
    def _():
        pltpu.sync_copy(zbuf.at[pl.ds(0, _LAST)],
                        acc_sh.at[pl.ds(_SPR * _NS, _LAST)])


def _writeback_stripes(acc_sh, out_hbm, c, s):
    row_base = pl.multiple_of(s * _SPR, 8)
    _stripe_copy(lambda o, n: acc_sh.at[pl.ds(row_base + o, n)],
                 lambda o, n: out_hbm.at[c, pl.ds(row_base + o, n)], _SPR)

    @pl.when(s == _NS - 1)
    def _():
        pltpu.sync_copy(acc_sh.at[pl.ds(_SPR * _NS, _N - _SPR * _NS)],
                        out_hbm.at[c, pl.ds(_SPR * _NS, _N - _SPR * _NS)])


def _spmm_body(src_hbm, dst_hbm, table_hbm, out_hbm,
               rows_a, rows_b,
               acc_sh, sem_a, sem_b):
    c = lax.axis_index("c")
    s = lax.axis_index("s")
    wid = s * _NC + c

    _fill(rows_a, _GSZ, _D, 0.0)
    _zero_stripes(acc_sh, rows_a, s)
    plsc.subcore_barrier()

    def body(siA, diA, siB, diB):
        cpA = pltpu.async_copy(table_hbm.at[siA.at[0]], rows_a, sem_a)
        cpB = pltpu.async_copy(table_hbm.at[siB.at[0]], rows_b, sem_b)
        cpA.wait()
        pltpu.sync_copy(rows_a, acc_sh.at[diA.at[0]], add=True)
        cpB.wait()
        pltpu.sync_copy(rows_b, acc_sh.at[diB.at[0]], add=True)

    gA = lambda k: (0, wid * _GPT + 2 * k)
    gB = lambda k: (0, wid * _GPT + 2 * k + 1)
    pltpu.emit_pipeline(
        body,
        grid=(_GPT // 2,),
        in_specs=[pl.BlockSpec((1, _GSZ), gA),
                  pl.BlockSpec((1, _GSZ), gA),
                  pl.BlockSpec((1, _GSZ), gB),
                  pl.BlockSpec((1, _GSZ), gB)],
        out_specs=[],
        dimension_semantics=(pltpu.ARBITRARY,),
    )(src_hbm, dst_hbm, src_hbm, dst_hbm)

    plsc.subcore_barrier()
    _writeback_stripes(acc_sh, out_hbm, c, s)


@functools.partial(
    pl.kernel, mesh=_mesh,
    out_type=jax.ShapeDtypeStruct((_NC, _N, _D), jnp.float32),
    scratch_types=[
        pltpu.VMEM((_GSZ, _D), jnp.float32),
        pltpu.VMEM((_GSZ, _D), jnp.float32),
        pltpu.VMEM_SHARED((_NA, _D), jnp.float32),
        pltpu.SemaphoreType.DMA,
        pltpu.SemaphoreType.DMA,
    ],
)
def _sc_spmm(*args):
    _spmm_body(*args)


def _hist_body(idx_hbm, out_hbm, buf, hist_sh):
    c = lax.axis_index("c")
    s = lax.axis_index("s")
    wid = s * _NC + c

    _fill(buf, _GSZ, _D, 0.0)
    _zero_stripes(hist_sh, buf, s)
    _fill(buf, _GSZ, _D, 1.0)
    plsc.subcore_barrier()

    def body(i_vmem):
        pltpu.sync_copy(buf, hist_sh.at[i_vmem.at[0]], add=True)

    gidx = lambda k: (0, wid * _GPT + k)
    pltpu.emit_pipeline(
        body,
        grid=(_GPT,),
        in_specs=[pl.BlockSpec((1, _GSZ), gidx)],
        out_specs=[],
        dimension_semantics=(pltpu.ARBITRARY,),
    )(idx_hbm)

    plsc.subcore_barrier()
    _writeback_stripes(hist_sh, out_hbm, c, s)


@functools.partial(
    pl.kernel, mesh=_mesh,
    out_type=jax.ShapeDtypeStruct((_NC, _N, _D), jnp.float32),
    scratch_types=[
        pltpu.VMEM((_GSZ, _D), jnp.float32),
        pltpu.VMEM_SHARED((_NA, _D), jnp.float32),
    ],
)
def _sc_hist(*args):
    _hist_body(*args)


_BLK = 2000


def _mm_body(x_ref, w_ref, o_ref):
    o_ref[...] = jnp.dot(x_ref[...], w_ref[...],
                         preferred_element_type=jnp.float32)


def _tc_matmul(x, w):
    return pl.pallas_call(
        _mm_body,
        grid=(_N // _BLK,),
        in_specs=[pl.BlockSpec((_BLK, _D), lambda i: (i, 0)),
                  pl.BlockSpec((_D, _D), lambda i: (0, 0))],
        out_specs=pl.BlockSpec((_BLK, _D), lambda i: (i, 0)),
        out_shape=jax.ShapeDtypeStruct((_N, _D), jnp.float32),
    )(x, w)


def _recip_deg(h_ref):
    hv = h_ref[...]
    d = hv[0, :, 0:1] + hv[1, :, 0:1]
    return jnp.where(d > 0, 1.0 / d, 0.0)


def _scale_body(p_ref, h_ref, o_ref):
    o_ref[...] = (p_ref[0] + p_ref[1]) * _recip_deg(h_ref)


def _tc_scale(parts, hist):
    return pl.pallas_call(
        _scale_body,
        grid=(_N // _BLK,),
        in_specs=[pl.BlockSpec((_NC, _BLK, _D), lambda i: (0, i, 0)),
                  pl.BlockSpec((_NC, _BLK, _D), lambda i: (0, i, 0))],
        out_specs=pl.BlockSpec((_BLK, _D), lambda i: (i, 0)),
        out_shape=jax.ShapeDtypeStruct((_N, _D), jnp.float32),
    )(parts, hist)


def _mid_body(p_ref, h_ref, b_ref, w_ref, o_ref):
    acc = (p_ref[0] + p_ref[1]) * _recip_deg(h_ref) + b_ref[...]
    hmid = jnp.maximum(acc, 0.0)
    o_ref[...] = jnp.dot(hmid, w_ref[...], preferred_element_type=jnp.float32)


def _tc_mid(parts, hist, b, w):
    return pl.pallas_call(
        _mid_body,
        grid=(_N // _BLK,),
        in_specs=[pl.BlockSpec((_NC, _BLK, _D), lambda i: (0, i, 0)),
                  pl.BlockSpec((_NC, _BLK, _D), lambda i: (0, i, 0)),
                  pl.BlockSpec((1, _D), lambda i: (0, 0)),
                  pl.BlockSpec((_D, _D), lambda i: (0, 0))],
        out_specs=pl.BlockSpec((_BLK, _D), lambda i: (i, 0)),
        out_shape=jax.ShapeDtypeStruct((_N, _D), jnp.float32),
    )(parts, hist, b, w)


def _fin_body(p_ref, h_ref, b_ref, o_ref):
    o_ref[...] = (p_ref[0] + p_ref[1]) * _recip_deg(h_ref) + b_ref[...]


def _tc_final(parts, hist, b):
    return pl.pallas_call(
        _fin_body,
        grid=(_N // _BLK,),
        in_specs=[pl.BlockSpec((_NC, _BLK, _D), lambda i: (0, i, 0)),
                  pl.BlockSpec((_NC, _BLK, _D), lambda i: (0, i, 0)),
                  pl.BlockSpec((1, _D), lambda i: (0, 0))],
        out_specs=pl.BlockSpec((_BLK, _D), lambda i: (i, 0)),
        out_shape=jax.ShapeDtypeStruct((_N, _D), jnp.float32),
    )(parts, hist, b)


def kernel(x, edge_index, W1, b1, W2, b2):
    npad = _PP - _P
    src_pad = (jnp.arange(npad, dtype=jnp.int32) * 61) % _N
    dst_pad = _N + (jnp.arange(npad, dtype=jnp.int32) % _NPAD)
    node_g = jnp.concatenate([edge_index[0], src_pad])
    hedge_g = jnp.concatenate([edge_index[1], src_pad])
    node_d = jnp.concatenate([edge_index[0], dst_pad])
    hedge_d = jnp.concatenate([edge_index[1], dst_pad])
    b1r = b1.reshape(1, _D)
    b2r = b2.reshape(1, _D)

    node_g = node_g.reshape(1, _PP)
    hedge_g = hedge_g.reshape(1, _PP)
    node_d = node_d.reshape(1, _PP)
    hedge_d = hedge_d.reshape(1, _PP)

    hist_n = _sc_hist(node_d)
    dep = (hist_n[0, 0, 0] * 0.0).astype(jnp.int32)
    hist_e = _sc_hist(hedge_d + dep)

    xw1 = _tc_matmul(x, W1)
    xw1 = xw1 + 0.0 * hist_e[0, 0, 0]
    ep1 = _sc_spmm(node_g, hedge_d, xw1)
    ef1 = _tc_scale(ep1, hist_e)
    np1 = _sc_spmm(hedge_g, node_d, ef1)
    xw2 = _tc_mid(np1, hist_n, b1r, W2)
    ep2 = _sc_spmm(node_g, hedge_d, xw2)
    ef2 = _tc_scale(ep2, hist_e)
    np2 = _sc_spmm(hedge_g, node_d, ef2)
    out = _tc_final(np2, hist_n, b2r)
    return out
